# baseline (speedup 1.0000x reference)
import functools

import jax
import jax.numpy as jnp
from jax import lax
from jax.experimental import pallas as pl
from jax.experimental.pallas import tpu as pltpu
from jax.experimental.pallas import tpu_sc as plsc

_BATCH = 20
_SEQ = 16384
_DIM = 64

_INFO = plsc.get_sparse_core_info()
_NC = _INFO.num_cores
_NS = _INFO.num_subcores
_NW = _NC * _NS

_N = _BATCH * _SEQ
_PER_W = _N // _NW
_G = 128
_NG = _PER_W // _G
_NBUF = 8


def _body(tbl, idx_hbm, out_hbm, idx_v, rows, gsems):
    c = lax.axis_index("c")
    s = lax.axis_index("s")
    wid = s * _NC + c
    base = wid * _PER_W

    pltpu.sync_copy(idx_hbm.at[wid], idx_v)

    def _wait(j, b):
        pltpu.make_async_copy(tbl.at[idx_v.at[j]], rows.at[b], gsems.at[b]).wait()

    for b in range(_NBUF):
        pltpu.async_copy(tbl.at[idx_v.at[b]], rows.at[b], gsems.at[b])

    def _group(g, carry):
        for b in range(_NBUF):
            j = g * _NBUF + b
            _wait(j, b)
            pltpu.sync_copy(rows.at[b], out_hbm.at[pl.ds(base + j * _G, _G)])
            pltpu.async_copy(tbl.at[idx_v.at[j + _NBUF]], rows.at[b], gsems.at[b])
        return carry

    lax.fori_loop(0, _NG // _NBUF - 1, _group, 0)

    for b in range(_NBUF):
        j = _NG - _NBUF + b
        _wait(j, b)
        pltpu.sync_copy(rows.at[b], out_hbm.at[pl.ds(base + j * _G, _G)])


@jax.jit
def _gather(table, idx):
    mesh = plsc.VectorSubcoreMesh(core_axis_name="c", subcore_axis_name="s")
    run = pl.kernel(
        _body,
        out_type=jax.ShapeDtypeStruct((_N, _DIM), jnp.float32),
        mesh=mesh,
        scratch_types=[
            pltpu.VMEM((_NG, _G), jnp.int32),
            pltpu.VMEM((_NBUF, _G, _DIM), jnp.float32),
            pltpu.SemaphoreType.DMA((_NBUF,)),
        ],
        compiler_params=pltpu.CompilerParams(use_tc_tiling_on_sc=False),
    )
    return run(table, idx)


def kernel(inputs, embedding_weight):
    idx = inputs.reshape(_NW, _NG, _G).astype(jnp.int32)
    rows = _gather(embedding_weight, idx)
    return rows.reshape(_BATCH, -1)


# speedup vs baseline: 2.5337x; 2.5337x over previous
import functools

import jax
import jax.numpy as jnp
from jax import lax
from jax.experimental import pallas as pl
from jax.experimental.pallas import tpu as pltpu
from jax.experimental.pallas import tpu_sc as plsc

_BATCH = 20
_SEQ = 16384
_DIM = 64

_INFO = plsc.get_sparse_core_info()
_NC = _INFO.num_cores
_NS = _INFO.num_subcores
_NW = _NC * _NS

_N = _BATCH * _SEQ
_PER_W = _N // _NW
_G = 128
_H = _G // 2
_NG = _PER_W // _G
_NBUF = 8


def _body(tbl, idxe_hbm, idxo_hbm, out_hbm, idxe_v, idxo_v, rowse, rowso, gsems):
    c = lax.axis_index("c")
    s = lax.axis_index("s")
    wid = s * _NC + c
    base = wid * (_PER_W // 2)

    pltpu.sync_copy(idxe_hbm.at[wid], idxe_v)
    pltpu.sync_copy(idxo_hbm.at[wid], idxo_v)

    def _fire(j, b):
        pltpu.async_copy(tbl.at[idxe_v.at[j]], rowse.at[b], gsems.at[b])
        pltpu.async_copy(tbl.at[idxo_v.at[j]], rowso.at[b], gsems.at[b])

    def _wait(j, b):
        pltpu.make_async_copy(tbl.at[idxe_v.at[j]], rowse.at[b],
                              gsems.at[b]).wait()
        pltpu.make_async_copy(tbl.at[idxo_v.at[j]], rowso.at[b],
                              gsems.at[b]).wait()

    def _store(j, b):
        r0 = base + j * _H
        pltpu.sync_copy(rowse.at[b], out_hbm.at[pl.ds(r0, _H), pl.ds(0, _DIM)])
        pltpu.sync_copy(rowso.at[b], out_hbm.at[pl.ds(r0, _H), pl.ds(_DIM, _DIM)])

    for b in range(_NBUF):
        _fire(b, b)

    def _group(g, carry):
        for b in range(_NBUF):
            j = g * _NBUF + b
            _wait(j, b)
            _store(j, b)
            _fire(j + _NBUF, b)
        return carry

    lax.fori_loop(0, _NG // _NBUF - 1, _group, 0)

    for b in range(_NBUF):
        j = _NG - _NBUF + b
        _wait(j, b)
        _store(j, b)


@jax.jit
def _gather(table, idxe, idxo):
    mesh = plsc.VectorSubcoreMesh(core_axis_name="c", subcore_axis_name="s")
    run = pl.kernel(
        _body,
        out_type=jax.ShapeDtypeStruct((_N // 2, 2 * _DIM), jnp.float32),
        mesh=mesh,
        scratch_types=[
            pltpu.VMEM((_NG, _H), jnp.int32),
            pltpu.VMEM((_NG, _H), jnp.int32),
            pltpu.VMEM((_NBUF, _H, _DIM), jnp.float32),
            pltpu.VMEM((_NBUF, _H, _DIM), jnp.float32),
            pltpu.SemaphoreType.DMA((_NBUF,)),
        ],
        compiler_params=pltpu.CompilerParams(use_tc_tiling_on_sc=False),
    )
    return run(table, idxe, idxo)


def kernel(inputs, embedding_weight):
    idx = inputs.reshape(_NW, _NG, _H, 2).astype(jnp.int32)
    rows = _gather(embedding_weight, idx[..., 0], idx[..., 1])
    return rows.reshape(_BATCH, -1)
